# Initial kernel scaffold; baseline (speedup 1.0000x reference)
import jax
import jax.numpy as jnp
from jax.experimental import pallas as pl


def kernel(inputs, labels, neg_samples, softmax_w, softmax_b):
    raise NotImplementedError("write your pallas kernel here")


# speedup vs baseline: 298.3040x; 298.3040x over previous
import functools

import jax
import jax.numpy as jnp
from jax import lax
from jax.experimental import pallas as pl
from jax.experimental.pallas import tpu as pltpu
from jax.experimental.pallas import tpu_sc as plsc

_N = 4096 * 50
_D = 64
_S = 100
_V = 1000000
_NW = 32
_PER_W = _N // _NW
_C = 128
_CHUNKS = _PER_W // _C
_G = _C // 16


def _sc_body(wb_hbm, neg_hbm, lab_hbm, xs_hbm, se_hbm, tl_hbm,
             negc_v, gat_v, lab_v, labg_v, xs_v, se_v, tl_v, sem):
    wid = lax.axis_index("s") * 2 + lax.axis_index("c")
    base = wid * _PER_W
    iota = lax.iota(jnp.int32, 16)
    himask = jnp.int32(-65536)

    def chunk_body(ci, carry):
        col0 = base + ci * _C
        pltpu.sync_copy(neg_hbm.at[pl.ds(col0 * _S, _C * _S)], negc_v)
        pltpu.sync_copy(lab_hbm.at[pl.ds(col0, _C)], lab_v)
        pltpu.sync_copy(xs_hbm.at[pl.ds(col0, _C)], xs_v)
        cp = pltpu.async_copy(wb_hbm.at[negc_v], gat_v, sem)
        cp2 = pltpu.async_copy(wb_hbm.at[lab_v], labg_v, sem)
        cp.wait()
        cp2.wait()

        xs = [xs_v[pl.ds(g * 16, 16)] for g in range(_G)]
        rows = [(iota + jnp.int32(g * 16)) * jnp.int32(_S) for g in range(_G)]

        def s_body(s, accs):
            out = []
            for g in range(_G):
                v = plsc.load_gather(gat_v, [rows[g] + s])
                w = lax.bitcast_convert_type(v & himask, jnp.float32)
                b = lax.bitcast_convert_type(lax.shift_left(v, jnp.int32(16)),
                                             jnp.float32)
                out.append(accs[g] + jnp.exp(w * xs[g] + b))
            return tuple(out)

        init = []
        for g in range(_G):
            lw = labg_v[pl.ds(g * 16, 16)]
            w = lax.bitcast_convert_type(lw & himask, jnp.float32)
            b = lax.bitcast_convert_type(lax.shift_left(lw, jnp.int32(16)),
                                         jnp.float32)
            tl = w * xs[g] + b
            tl_v[pl.ds(g * 16, 16)] = tl
            init.append(jnp.exp(tl))

        accs = lax.fori_loop(0, _S, s_body, tuple(init))
        for g in range(_G):
            se_v[pl.ds(g * 16, 16)] = accs[g]

        pltpu.sync_copy(se_v, se_hbm.at[pl.ds(col0, _C)])
        pltpu.sync_copy(tl_v, tl_hbm.at[pl.ds(col0, _C)])
        return carry

    lax.fori_loop(0, _CHUNKS, chunk_body, jnp.int32(0))


@jax.jit
def _sc_gather_expsum(wb, neg, lab, xs):
    mesh = plsc.VectorSubcoreMesh(core_axis_name="c", subcore_axis_name="s")
    f = pl.kernel(
        _sc_body,
        out_type=(jax.ShapeDtypeStruct((_N,), jnp.float32),
                  jax.ShapeDtypeStruct((_N,), jnp.float32)),
        mesh=mesh,
        compiler_params=pltpu.CompilerParams(needs_layout_passes=False),
        scratch_types=[
            pltpu.VMEM((_C * _S,), jnp.int32),
            pltpu.VMEM((_C * _S,), jnp.int32),
            pltpu.VMEM((_C,), jnp.int32),
            pltpu.VMEM((_C,), jnp.int32),
            pltpu.VMEM((_C,), jnp.float32),
            pltpu.VMEM((_C,), jnp.float32),
            pltpu.VMEM((_C,), jnp.float32),
            pltpu.SemaphoreType.DMA,
        ],
    )
    return f(wb, neg, lab, xs)


def _xsum_body(x_ref, o_ref):
    o_ref[...] = jnp.sum(x_ref[...], axis=2)


def _loss_body(se_ref, tl_ref, lab_ref, o_ref):
    mask = (lab_ref[...] != 0).astype(jnp.float32)
    ce = jnp.log(se_ref[...]) - tl_ref[...]
    o_ref[...] = (jnp.sum(ce * mask) / jnp.sum(mask)).reshape(1, 1)


def kernel(inputs, labels, neg_samples, softmax_w, softmax_b):
    x3 = inputs.reshape(_N // 128, 128, _D)
    lab = labels.reshape(-1).astype(jnp.int32)
    neg = neg_samples.reshape(-1).astype(jnp.int32)

    w16 = lax.bitcast_convert_type(
        softmax_w.reshape(-1).astype(jnp.bfloat16), jnp.uint16).astype(jnp.uint32)
    b16 = lax.bitcast_convert_type(
        softmax_b.astype(jnp.bfloat16), jnp.uint16).astype(jnp.uint32)
    wb = lax.bitcast_convert_type((w16 << 16) | b16, jnp.int32)

    xsum2d = pl.pallas_call(
        _xsum_body,
        grid=(_N // (128 * 32),),
        in_specs=[pl.BlockSpec((32, 128, _D), lambda i: (i, 0, 0))],
        out_specs=pl.BlockSpec((32, 128), lambda i: (i, 0)),
        out_shape=jax.ShapeDtypeStruct((_N // 128, 128), jnp.float32),
    )(x3)

    se, tl = _sc_gather_expsum(wb, neg, lab, xsum2d.reshape(-1))

    loss = pl.pallas_call(
        _loss_body,
        in_specs=[pl.BlockSpec((_N // 128, 128), lambda: (0, 0)),
                  pl.BlockSpec((_N // 128, 128), lambda: (0, 0)),
                  pl.BlockSpec((_N // 128, 128), lambda: (0, 0))],
        out_specs=pl.BlockSpec((1, 1), lambda: (0, 0)),
        out_shape=jax.ShapeDtypeStruct((1, 1), jnp.float32),
    )(se.reshape(_N // 128, 128), tl.reshape(_N // 128, 128),
      lab.reshape(_N // 128, 128))

    return loss.reshape(())


# speedup vs baseline: 522.3797x; 1.7512x over previous
import functools

import jax
import jax.numpy as jnp
from jax import lax
from jax.experimental import pallas as pl
from jax.experimental.pallas import tpu as pltpu
from jax.experimental.pallas import tpu_sc as plsc

_N = 4096 * 50
_D = 64
_S = 100
_V = 1000000
_NW = 32
_PER_W = _N // _NW
_C = 128
_CHUNKS = _PER_W // _C
_G = _C // 16


def _sc_body(wb_hbm, neg_hbm, lab_hbm, xs_hbm, se_hbm, tl_hbm,
             wb_sp, negc_v, gat_v, lab_v, labg_v, xs_v, se_v, tl_v, sem):
    wid = lax.axis_index("s") * 2 + lax.axis_index("c")
    base = wid * _PER_W
    iota = lax.iota(jnp.int32, 16)
    himask = jnp.int32(-65536)

    @pl.when(lax.axis_index("s") == 0)
    def _load_table():
        pltpu.sync_copy(wb_hbm, wb_sp)

    plsc.subcore_barrier()

    def chunk_body(ci, carry):
        col0 = base + ci * _C
        pltpu.sync_copy(neg_hbm.at[pl.ds(col0 * _S, _C * _S)], negc_v)
        pltpu.sync_copy(lab_hbm.at[pl.ds(col0, _C)], lab_v)
        pltpu.sync_copy(xs_hbm.at[pl.ds(col0, _C)], xs_v)
        cp = pltpu.async_copy(wb_sp.at[negc_v], gat_v, sem)
        cp2 = pltpu.async_copy(wb_sp.at[lab_v], labg_v, sem)
        cp.wait()
        cp2.wait()

        xs = [xs_v[pl.ds(g * 16, 16)] for g in range(_G)]
        rows = [(iota + jnp.int32(g * 16)) * jnp.int32(_S) for g in range(_G)]

        def s_body(s, accs):
            out = []
            for g in range(_G):
                v = plsc.load_gather(gat_v, [rows[g] + s])
                w = lax.bitcast_convert_type(v & himask, jnp.float32)
                b = lax.bitcast_convert_type(lax.shift_left(v, jnp.int32(16)),
                                             jnp.float32)
                out.append(accs[g] + jnp.exp(w * xs[g] + b))
            return tuple(out)

        init = []
        for g in range(_G):
            lw = labg_v[pl.ds(g * 16, 16)]
            w = lax.bitcast_convert_type(lw & himask, jnp.float32)
            b = lax.bitcast_convert_type(lax.shift_left(lw, jnp.int32(16)),
                                         jnp.float32)
            tl = w * xs[g] + b
            tl_v[pl.ds(g * 16, 16)] = tl
            init.append(jnp.exp(tl))

        accs = lax.fori_loop(0, _S, s_body, tuple(init))
        for g in range(_G):
            se_v[pl.ds(g * 16, 16)] = accs[g]

        pltpu.sync_copy(se_v, se_hbm.at[pl.ds(col0, _C)])
        pltpu.sync_copy(tl_v, tl_hbm.at[pl.ds(col0, _C)])
        return carry

    lax.fori_loop(0, _CHUNKS, chunk_body, jnp.int32(0))


@jax.jit
def _sc_gather_expsum(wb, neg, lab, xs):
    mesh = plsc.VectorSubcoreMesh(core_axis_name="c", subcore_axis_name="s")
    f = pl.kernel(
        _sc_body,
        out_type=(jax.ShapeDtypeStruct((_N,), jnp.float32),
                  jax.ShapeDtypeStruct((_N,), jnp.float32)),
        mesh=mesh,
        compiler_params=pltpu.CompilerParams(needs_layout_passes=False),
        scratch_types=[
            pltpu.VMEM_SHARED((_V,), jnp.int32),
            pltpu.VMEM((_C * _S,), jnp.int32),
            pltpu.VMEM((_C * _S,), jnp.int32),
            pltpu.VMEM((_C,), jnp.int32),
            pltpu.VMEM((_C,), jnp.int32),
            pltpu.VMEM((_C,), jnp.float32),
            pltpu.VMEM((_C,), jnp.float32),
            pltpu.VMEM((_C,), jnp.float32),
            pltpu.SemaphoreType.DMA,
        ],
    )
    return f(wb, neg, lab, xs)


def _xsum_body(x_ref, o_ref):
    o_ref[...] = jnp.sum(x_ref[...], axis=2)


def _loss_body(se_ref, tl_ref, lab_ref, o_ref):
    mask = (lab_ref[...] != 0).astype(jnp.float32)
    ce = jnp.log(se_ref[...]) - tl_ref[...]
    o_ref[...] = (jnp.sum(ce * mask) / jnp.sum(mask)).reshape(1, 1)


def kernel(inputs, labels, neg_samples, softmax_w, softmax_b):
    x3 = inputs.reshape(_N // 128, 128, _D)
    lab = labels.reshape(-1).astype(jnp.int32)
    neg = neg_samples.reshape(-1).astype(jnp.int32)

    w16 = lax.bitcast_convert_type(
        softmax_w.reshape(-1).astype(jnp.bfloat16), jnp.uint16).astype(jnp.uint32)
    b16 = lax.bitcast_convert_type(
        softmax_b.astype(jnp.bfloat16), jnp.uint16).astype(jnp.uint32)
    wb = lax.bitcast_convert_type((w16 << 16) | b16, jnp.int32)

    xsum2d = pl.pallas_call(
        _xsum_body,
        grid=(_N // (128 * 32),),
        in_specs=[pl.BlockSpec((32, 128, _D), lambda i: (i, 0, 0))],
        out_specs=pl.BlockSpec((32, 128), lambda i: (i, 0)),
        out_shape=jax.ShapeDtypeStruct((_N // 128, 128), jnp.float32),
    )(x3)

    se, tl = _sc_gather_expsum(wb, neg, lab, xsum2d.reshape(-1))

    loss = pl.pallas_call(
        _loss_body,
        in_specs=[pl.BlockSpec((_N // 128, 128), lambda: (0, 0)),
                  pl.BlockSpec((_N // 128, 128), lambda: (0, 0)),
                  pl.BlockSpec((_N // 128, 128), lambda: (0, 0))],
        out_specs=pl.BlockSpec((1, 1), lambda: (0, 0)),
        out_shape=jax.ShapeDtypeStruct((1, 1), jnp.float32),
    )(se.reshape(_N // 128, 128), tl.reshape(_N // 128, 128),
      lab.reshape(_N // 128, 128))

    return loss.reshape(())


# speedup vs baseline: 755.8244x; 1.4469x over previous
import functools

import jax
import jax.numpy as jnp
from jax import lax
from jax.experimental import pallas as pl
from jax.experimental.pallas import tpu as pltpu
from jax.experimental.pallas import tpu_sc as plsc

_N = 4096 * 50
_D = 64
_S = 100
_V = 1000000
_NW = 32
_PER_W = _N // _NW
_C = 128
_CHUNKS = _PER_W // _C
_G = _C // 16


def _sc_body(wb_hbm, neg_hbm, lab_hbm, xs_hbm, se_hbm, tl_hbm,
             wb_sp, negc0, negc1, gat0, gat1, lab0, lab1, labg0, labg1,
             xs0, xs1, se0, se1, tl0, tl1, sem0, sem1):
    wid = lax.axis_index("s") * 2 + lax.axis_index("c")
    base = wid * _PER_W
    iota = lax.iota(jnp.int32, 16)
    himask = jnp.int32(-65536)

    @pl.when(lax.axis_index("s") == 0)
    def _load_table():
        pltpu.sync_copy(wb_hbm, wb_sp)

    plsc.subcore_barrier()

    bufs = ((negc0, gat0, lab0, labg0, xs0, se0, tl0, sem0),
            (negc1, gat1, lab1, labg1, xs1, se1, tl1, sem1))

    def stage_fire(ci, b):
        negc, gat, lab, labg, xs, _, _, sem = bufs[b]
        col0 = base + ci * _C
        pltpu.sync_copy(neg_hbm.at[pl.ds(col0 * _S, _C * _S)], negc)
        pltpu.sync_copy(lab_hbm.at[pl.ds(col0, _C)], lab)
        pltpu.sync_copy(xs_hbm.at[pl.ds(col0, _C)], xs)
        pltpu.async_copy(wb_sp.at[negc], gat, sem)
        pltpu.async_copy(wb_sp.at[lab], labg, sem)

    def compute(ci, b):
        negc, gat, lab, labg, xs_v, se_v, tl_v, sem = bufs[b]
        pltpu.make_async_copy(wb_sp.at[negc], gat, sem).wait()
        pltpu.make_async_copy(wb_sp.at[lab], labg, sem).wait()

        xs = [xs_v[pl.ds(g * 16, 16)] for g in range(_G)]
        rows = [(iota + jnp.int32(g * 16)) * jnp.int32(_S) for g in range(_G)]

        def s_body(s, accs):
            out = []
            for g in range(_G):
                v = plsc.load_gather(gat, [rows[g] + s])
                w = lax.bitcast_convert_type(v & himask, jnp.float32)
                b_ = lax.bitcast_convert_type(lax.shift_left(v, jnp.int32(16)),
                                              jnp.float32)
                out.append(accs[g] + jnp.exp(w * xs[g] + b_))
            return tuple(out)

        init = []
        for g in range(_G):
            lw = labg[pl.ds(g * 16, 16)]
            w = lax.bitcast_convert_type(lw & himask, jnp.float32)
            b_ = lax.bitcast_convert_type(lax.shift_left(lw, jnp.int32(16)),
                                          jnp.float32)
            tl = w * xs[g] + b_
            tl_v[pl.ds(g * 16, 16)] = tl
            init.append(jnp.exp(tl))

        accs = lax.fori_loop(0, _S, s_body, tuple(init))
        for g in range(_G):
            se_v[pl.ds(g * 16, 16)] = accs[g]

        col0 = base + ci * _C
        pltpu.sync_copy(se_v, se_hbm.at[pl.ds(col0, _C)])
        pltpu.sync_copy(tl_v, tl_hbm.at[pl.ds(col0, _C)])

    stage_fire(jnp.int32(0), 0)

    def pair_body(i, carry):
        c0 = i * 2
        stage_fire(c0 + 1, 1)
        compute(c0, 0)

        @pl.when(c0 + 2 < _CHUNKS)
        def _():
            stage_fire(c0 + 2, 0)

        compute(c0 + 1, 1)
        return carry

    lax.fori_loop(0, _CHUNKS // 2, pair_body, jnp.int32(0))


@jax.jit
def _sc_gather_expsum(wb, neg, lab, xs):
    mesh = plsc.VectorSubcoreMesh(core_axis_name="c", subcore_axis_name="s")
    dbl = lambda t: (t, t)
    f = pl.kernel(
        _sc_body,
        out_type=(jax.ShapeDtypeStruct((_N,), jnp.float32),
                  jax.ShapeDtypeStruct((_N,), jnp.float32)),
        mesh=mesh,
        compiler_params=pltpu.CompilerParams(needs_layout_passes=False),
        scratch_types=[
            pltpu.VMEM_SHARED((_V,), jnp.int32),
            *dbl(pltpu.VMEM((_C * _S,), jnp.int32)),
            *dbl(pltpu.VMEM((_C * _S,), jnp.int32)),
            *dbl(pltpu.VMEM((_C,), jnp.int32)),
            *dbl(pltpu.VMEM((_C,), jnp.int32)),
            *dbl(pltpu.VMEM((_C,), jnp.float32)),
            *dbl(pltpu.VMEM((_C,), jnp.float32)),
            *dbl(pltpu.VMEM((_C,), jnp.float32)),
            *dbl(pltpu.SemaphoreType.DMA),
        ],
    )
    return f(wb, neg, lab, xs)


def _xsum_body(x_ref, o_ref):
    o_ref[...] = jnp.sum(x_ref[...], axis=2)


def _loss_body(se_ref, tl_ref, lab_ref, o_ref):
    mask = (lab_ref[...] != 0).astype(jnp.float32)
    ce = jnp.log(se_ref[...]) - tl_ref[...]
    o_ref[...] = (jnp.sum(ce * mask) / jnp.sum(mask)).reshape(1, 1)


def kernel(inputs, labels, neg_samples, softmax_w, softmax_b):
    lab = labels.reshape(-1).astype(jnp.int32)
    neg = neg_samples.reshape(-1).astype(jnp.int32)

    w16 = lax.bitcast_convert_type(
        softmax_w.reshape(-1).astype(jnp.bfloat16), jnp.uint16).astype(jnp.uint32)
    b16 = lax.bitcast_convert_type(
        softmax_b.astype(jnp.bfloat16), jnp.uint16).astype(jnp.uint32)
    wb = lax.bitcast_convert_type((w16 << 16) | b16, jnp.int32)

    nb, nt = inputs.shape[0], inputs.shape[1]
    xsum2d = pl.pallas_call(
        _xsum_body,
        grid=(nb // 128,),
        in_specs=[pl.BlockSpec((128, nt, _D), lambda i: (i, 0, 0))],
        out_specs=pl.BlockSpec((128, nt), lambda i: (i, 0)),
        out_shape=jax.ShapeDtypeStruct((nb, nt), jnp.float32),
    )(inputs)

    se, tl = _sc_gather_expsum(wb, neg, lab, xsum2d.reshape(-1))

    loss = pl.pallas_call(
        _loss_body,
        in_specs=[pl.BlockSpec((_N // 128, 128), lambda: (0, 0)),
                  pl.BlockSpec((_N // 128, 128), lambda: (0, 0)),
                  pl.BlockSpec((_N // 128, 128), lambda: (0, 0))],
        out_specs=pl.BlockSpec((1, 1), lambda: (0, 0)),
        out_shape=jax.ShapeDtypeStruct((1, 1), jnp.float32),
    )(se.reshape(_N // 128, 128), tl.reshape(_N // 128, 128),
      lab.reshape(_N // 128, 128))

    return loss.reshape(())


# speedup vs baseline: 882.9948x; 1.1683x over previous
import functools

import jax
import jax.numpy as jnp
from jax import lax
from jax.experimental import pallas as pl
from jax.experimental.pallas import tpu as pltpu
from jax.experimental.pallas import tpu_sc as plsc

_B = 4096
_T = 50
_N = _B * _T
_D = 64
_S = 100
_V = 1000000
_NW = 32
_SB = 2
_NP = _S // _SB
_NTASK = _T * _NP
_KMAX = (_NTASK + _NW - 1) // _NW
_NPAD = 56
_MSHIFT = (1 << 16) // _NP + 1


def _tp(tau):
    t = lax.shift_right_logical(tau * jnp.int32(_MSHIFT), jnp.int32(16))
    return t, tau - jnp.int32(_NP) * t


def _sc_body(wb_hbm, neg_hbm, lab_hbm, xs_hbm, sep_hbm, tl_hbm,
             wb_sp, negc0a, negc0b, negc1a, negc1b, gat0a, gat0b,
             gat1a, gat1b, xsv0, xsv1, sev0, sev1,
             labv, labgv, tlv, sem0, sem1):
    wid = lax.axis_index("s") * 2 + lax.axis_index("c")
    himask = jnp.int32(-65536)

    @pl.when(lax.axis_index("s") == 0)
    def _load_table():
        pltpu.sync_copy(wb_hbm, wb_sp)

    plsc.subcore_barrier()

    bufs = (((negc0a, negc0b), (gat0a, gat0b), xsv0, sev0, sem0),
            ((negc1a, negc1b), (gat1a, gat1b), xsv1, sev1, sem1))

    def stage_fire(tau, b):
        negc, gat, xsv, _, sem = bufs[b]
        t, p = _tp(tau)
        for sl in range(_SB):
            pltpu.sync_copy(neg_hbm.at[t, p * _SB + sl, :], negc[sl])
        pltpu.sync_copy(xs_hbm.at[t, :], xsv)
        for sl in range(_SB):
            pltpu.async_copy(wb_sp.at[negc[sl]], gat[sl], sem)

        @pl.when(p == 0)
        def _():
            pltpu.sync_copy(lab_hbm.at[t, :], labv)
            pltpu.async_copy(wb_sp.at[labv], labgv, sem)

    def unpack_w(v):
        return lax.bitcast_convert_type(v & himask, jnp.float32)

    def unpack_b(v):
        return lax.bitcast_convert_type(lax.shift_left(v, jnp.int32(16)),
                                        jnp.float32)

    def compute(tau, b):
        negc, gat, xsv, sev, sem = bufs[b]
        t, p = _tp(tau)
        for sl in range(_SB):
            pltpu.make_async_copy(wb_sp.at[negc[sl]], gat[sl], sem).wait()

        def expsum(bg, with_true):
            base = bg * 16
            xs = xsv[pl.ds(base, 16)]
            acc = jnp.zeros((16,), jnp.float32)
            for sl in range(_SB):
                v = gat[sl][pl.ds(base, 16)]
                acc = acc + jnp.exp(unpack_w(v) * xs + unpack_b(v))
            if with_true:
                lw = labgv[pl.ds(base, 16)]
                tl = unpack_w(lw) * xs + unpack_b(lw)
                tlv[pl.ds(base, 16)] = tl
                acc = acc + jnp.exp(tl)
            sev[pl.ds(base, 16)] = acc
            return 0

        @pl.when(p == 0)
        def _():
            pltpu.make_async_copy(wb_sp.at[labv], labgv, sem).wait()
            lax.fori_loop(0, _B // 16, lambda bg, c: expsum(bg, True), 0)
            pltpu.sync_copy(tlv, tl_hbm.at[t, :])

        @pl.when(p != 0)
        def _():
            lax.fori_loop(0, _B // 16, lambda bg, c: expsum(bg, False), 0)

        row = (p * jnp.int32(_T) + t) * jnp.int32(_B)
        pltpu.sync_copy(sev, sep_hbm.at[pl.ds(row, _B)])

    tau0 = wid
    stage_fire(tau0, 0)

    def pair_body(i, carry):
        tau_a = carry
        tau_b = tau_a + _NW
        tau_c = tau_a + 2 * _NW

        @pl.when(tau_b < _NTASK)
        def _():
            stage_fire(tau_b, 1)

        @pl.when(tau_a < _NTASK)
        def _():
            compute(tau_a, 0)

        @pl.when(tau_c < _NTASK)
        def _():
            stage_fire(tau_c, 0)

        @pl.when(tau_b < _NTASK)
        def _():
            compute(tau_b, 1)

        return tau_c

    lax.fori_loop(0, (_KMAX + 1) // 2, pair_body, tau0)


@jax.jit
def _sc_gather_expsum(wb, neg, lab, xs):
    mesh = plsc.VectorSubcoreMesh(core_axis_name="c", subcore_axis_name="s")
    dbl = lambda t: (t, t)
    f = pl.kernel(
        _sc_body,
        out_type=(jax.ShapeDtypeStruct((_NPAD * _T * _B,), jnp.float32),
                  jax.ShapeDtypeStruct((_T, _B), jnp.float32)),
        mesh=mesh,
        compiler_params=pltpu.CompilerParams(needs_layout_passes=False),
        scratch_types=[
            pltpu.VMEM_SHARED((_V,), jnp.int32),
            *(pltpu.VMEM((_B,), jnp.int32) for _ in range(4)),
            *(pltpu.VMEM((_B,), jnp.int32) for _ in range(4)),
            *dbl(pltpu.VMEM((_B,), jnp.float32)),
            *dbl(pltpu.VMEM((_B,), jnp.float32)),
            pltpu.VMEM((_B,), jnp.int32),
            pltpu.VMEM((_B,), jnp.int32),
            pltpu.VMEM((_B,), jnp.float32),
            *dbl(pltpu.SemaphoreType.DMA),
        ],
    )
    return f(wb, neg, lab, xs)


def _xsum_body(x_ref, o_ref):
    o_ref[...] = jnp.sum(x_ref[...], axis=1)


def _loss_body(sep_ref, tl_ref, lab_ref, o_ref, acc_ref):
    i = pl.program_id(0)

    @pl.when(i == 0)
    def _():
        acc_ref[...] = jnp.zeros_like(acc_ref)

    se = sep_ref[pl.ds(0, _T), :]
    for p in range(1, _NP):
        se = se + sep_ref[pl.ds(p * _T, _T), :]
    mask = (lab_ref[...] != 0).astype(jnp.float32)
    ce = jnp.log(se) - tl_ref[...]
    acc_ref[pl.ds(0, 1), :] += jnp.sum(ce * mask, axis=0, keepdims=True)
    acc_ref[pl.ds(1, 1), :] += jnp.sum(mask, axis=0, keepdims=True)

    @pl.when(i == pl.num_programs(0) - 1)
    def _():
        o_ref[...] = (jnp.sum(acc_ref[pl.ds(0, 1), :]) /
                      jnp.sum(acc_ref[pl.ds(1, 1), :])).reshape(1, 1)


def kernel(inputs, labels, neg_samples, softmax_w, softmax_b):
    x_t = jnp.transpose(inputs, (1, 2, 0))
    lab_t = jnp.transpose(labels.astype(jnp.int32))
    neg_t = jnp.transpose(neg_samples.astype(jnp.int32), (1, 2, 0))

    w16 = lax.bitcast_convert_type(
        softmax_w.reshape(-1).astype(jnp.bfloat16), jnp.uint16).astype(jnp.uint32)
    b16 = lax.bitcast_convert_type(
        softmax_b.astype(jnp.bfloat16), jnp.uint16).astype(jnp.uint32)
    wb = lax.bitcast_convert_type((w16 << 16) | b16, jnp.int32)

    xsum = pl.pallas_call(
        _xsum_body,
        grid=(8,),
        in_specs=[pl.BlockSpec((_T, _D, _B // 8), lambda i: (0, 0, i))],
        out_specs=pl.BlockSpec((_T, _B // 8), lambda i: (0, i)),
        out_shape=jax.ShapeDtypeStruct((_T, _B), jnp.float32),
    )(x_t)

    sep, tl = _sc_gather_expsum(wb, neg_t, lab_t, xsum)

    loss = pl.pallas_call(
        _loss_body,
        grid=(8,),
        in_specs=[pl.BlockSpec((_NPAD * _T, _B // 8), lambda i: (0, i)),
                  pl.BlockSpec((_T, _B // 8), lambda i: (0, i)),
                  pl.BlockSpec((_T, _B // 8), lambda i: (0, i))],
        out_specs=pl.BlockSpec((1, 1), lambda i: (0, 0)),
        out_shape=jax.ShapeDtypeStruct((1, 1), jnp.float32),
        scratch_shapes=[pltpu.VMEM((8, _B // 8), jnp.float32)],
    )(sep.reshape(_NPAD * _T, _B), tl, lab_t)

    return loss.reshape(())
